# baseline (speedup 1.0000x reference)
import functools
import jax
import jax.numpy as jnp
from jax.experimental import pallas as pl

_NU, _NI, _NB, _D = 50000, 40000, 20000, 64
_OFF_I = _NU
_OFF_U2 = _NU + _NI
_OFF_B = _OFF_U2 + _NU
_NTOT = _OFF_B + _NB
_NPAD = 163840
_EPS = 1e-8


def _scale_body(d_ref, o_ref):
    o_ref[...] = 1.0 / (jnp.sqrt(d_ref[...]) + _EPS)


def _inv_body(d_ref, o_ref):
    o_ref[...] = 1.0 / (d_ref[...] + _EPS)


def _ew_1d(body, x, rows=128):
    n = x.shape[0]
    assert (n // 128) % rows == 0, n
    x2 = x.reshape(n // 128, 128)
    out = pl.pallas_call(
        body,
        out_shape=jax.ShapeDtypeStruct((n // 128, 128), jnp.float32),
        grid=(n // 128 // rows,),
        in_specs=[pl.BlockSpec((rows, 128), lambda i: (i, 0))],
        out_specs=pl.BlockSpec((rows, 128), lambda i: (i, 0)),
    )(x2)
    return out.reshape(n)


def _mul_body(x_ref, s_ref, o_ref):
    o_ref[...] = x_ref[...] * s_ref[...]


def _rowscale(x, s, rows=512):
    n = x.shape[0]
    return pl.pallas_call(
        _mul_body,
        out_shape=jax.ShapeDtypeStruct((n, _D), jnp.float32),
        grid=(n // rows,),
        in_specs=[pl.BlockSpec((rows, _D), lambda i: (i, 0)),
                  pl.BlockSpec((rows, 1), lambda i: (i, 0))],
        out_specs=pl.BlockSpec((rows, _D), lambda i: (i, 0)),
    )(x, s.reshape(n, 1))


def _layer_body(inv_l, h_ref, s_ref, acc_ref, acc_o_ref, g_o_ref):
    s = s_ref[...]
    f = h_ref[...] * s * inv_l
    nrm = jnp.maximum(jnp.sqrt(jnp.sum(f * f, axis=1, keepdims=True)), 1e-12)
    acc_o_ref[...] = acc_ref[...] + f / nrm
    g_o_ref[...] = f * s


def _layer_update(h, s, acc, inv_l, rows=512):
    n = h.shape[0]
    acc_n, g_n = pl.pallas_call(
        functools.partial(_layer_body, inv_l),
        out_shape=(jax.ShapeDtypeStruct((n, _D), jnp.float32),
                   jax.ShapeDtypeStruct((n, _D), jnp.float32)),
        grid=(n // rows,),
        in_specs=[pl.BlockSpec((rows, _D), lambda i: (i, 0)),
                  pl.BlockSpec((rows, 1), lambda i: (i, 0)),
                  pl.BlockSpec((rows, _D), lambda i: (i, 0))],
        out_specs=(pl.BlockSpec((rows, _D), lambda i: (i, 0)),
                   pl.BlockSpec((rows, _D), lambda i: (i, 0))),
    )(h, s.reshape(n, 1), acc)
    return acc_n, g_n


def _segsum(g, src, dst, n_out):
    return jax.ops.segment_sum(g[src], dst, num_segments=n_out)


def kernel(users_feature, bundles_feature, items_feature, ui_u, ui_i, ub_u, ub_b, bi_b, bi_i):
    deg = jnp.concatenate([
        jnp.bincount(ui_u, length=_NU).astype(jnp.float32),
        jnp.bincount(ui_i, length=_NI).astype(jnp.float32),
        jnp.bincount(ub_u, length=_NU).astype(jnp.float32),
        jnp.bincount(ub_b, length=_NB).astype(jnp.float32),
        jnp.zeros((_NPAD - _NTOT,), jnp.float32),
    ])
    s_all = _ew_1d(_scale_body, deg)

    table0 = jnp.concatenate([
        users_feature, items_feature, users_feature, bundles_feature,
        jnp.zeros((_NPAD - _NTOT, _D), jnp.float32),
    ])
    acc = table0
    g = _rowscale(table0, s_all)

    src = jnp.concatenate([ui_i + _OFF_I, ui_u, ub_b + _OFF_B, ub_u + _OFF_U2])
    dst = jnp.concatenate([ui_u, ui_i + _OFF_I, ub_u + _OFF_U2, ub_b + _OFF_B])

    for i in range(2):
        h = _segsum(g, src, dst, _NPAD)
        acc, g = _layer_update(h, s_all, acc, 1.0 / (i + 2))

    bs = jnp.bincount(bi_b, length=_NB).astype(jnp.float32)
    bs = jnp.concatenate([bs, jnp.zeros((20480 - _NB,), jnp.float32)])
    inv_bs = _ew_1d(_inv_body, bs, rows=160)
    h_bi = _segsum(acc, bi_i + _OFF_I, bi_b, 20480)
    il_bundles = _rowscale(h_bi, inv_bs)[:_NB]

    return (acc[:_NU], acc[_OFF_U2:_OFF_U2 + _NU], il_bundles,
            acc[_OFF_B:_OFF_B + _NB])


# speedup vs baseline: 6.0701x; 6.0701x over previous
import functools
import jax
import jax.numpy as jnp
from jax import lax
from jax.experimental import pallas as pl
from jax.experimental.pallas import tpu as pltpu
from jax.experimental.pallas import tpu_sc as plsc

_NU, _NI, _NB, _D = 50000, 40000, 20000, 64
_OFF_I = _NU
_OFF_U2 = _NU + _NI
_OFF_B = _OFF_U2 + _NU
_NTOT = _OFF_B + _NB
_NPAD = 163840
_EPS = 1e-8
_SENT = 1 << 28
_B = 2048
_G = 128
_ZR = 512
_NSUB = 16
_NSLAB = 4
_W = 16

_DSEG0 = (0, 50048)
_DSEG1 = (0, 50048, 70080)
_DEG_ROWS = 98304
_DEG_RPS = _DEG_ROWS // _NSUB


def _pad_edges(x, blk=_B * _NSUB):
    e = x.shape[0]
    ep = ((e + blk - 1) // blk) * blk
    return jnp.concatenate([x, jnp.full((ep - e,), _SENT, jnp.int32)])


def _scale_body(d_ref, o_ref):
    o_ref[...] = 1.0 / (jnp.sqrt(d_ref[...]) + _EPS)


def _inv_body(d_ref, o_ref):
    o_ref[...] = 1.0 / (d_ref[...] + _EPS)


def _ew_1d(body, x, rows=128):
    n = x.shape[0]
    assert (n // 128) % rows == 0, n
    x2 = x.reshape(n // 128, 128)
    out = pl.pallas_call(
        body,
        out_shape=jax.ShapeDtypeStruct((n // 128, 128), jnp.float32),
        grid=(n // 128 // rows,),
        in_specs=[pl.BlockSpec((rows, 128), lambda i: (i, 0))],
        out_specs=pl.BlockSpec((rows, 128), lambda i: (i, 0)),
    )(x2)
    return out.reshape(n)


def _mul_body(x_ref, s_ref, o_ref):
    o_ref[...] = x_ref[...] * s_ref[...]


def _rowscale(x, s, rows=512):
    n = x.shape[0]
    return pl.pallas_call(
        _mul_body,
        out_shape=jax.ShapeDtypeStruct((n, _D), jnp.float32),
        grid=(n // rows,),
        in_specs=[pl.BlockSpec((rows, _D), lambda i: (i, 0)),
                  pl.BlockSpec((rows, 1), lambda i: (i, 0))],
        out_specs=pl.BlockSpec((rows, _D), lambda i: (i, 0)),
    )(x, s.reshape(n, 1))


def _layer_body(inv_l, h_ref, s_ref, acc_ref, acc_o_ref, g_o_ref):
    s = s_ref[...]
    f = h_ref[...] * s * inv_l
    nrm = jnp.maximum(jnp.sqrt(jnp.sum(f * f, axis=1, keepdims=True)), 1e-12)
    acc_o_ref[...] = acc_ref[...] + f / nrm
    g_o_ref[...] = f * s


def _layer_update(h, s, acc, inv_l, rows=512):
    n = h.shape[0]
    return pl.pallas_call(
        functools.partial(_layer_body, inv_l),
        out_shape=(jax.ShapeDtypeStruct((n, _D), jnp.float32),
                   jax.ShapeDtypeStruct((n, _D), jnp.float32)),
        grid=(n // rows,),
        in_specs=[pl.BlockSpec((rows, _D), lambda i: (i, 0)),
                  pl.BlockSpec((rows, 1), lambda i: (i, 0)),
                  pl.BlockSpec((rows, _D), lambda i: (i, 0))],
        out_specs=(pl.BlockSpec((rows, _D), lambda i: (i, 0)),
                   pl.BlockSpec((rows, _D), lambda i: (i, 0))),
    )(h, s.reshape(n, 1), acc)


def _spmm_body(phases, acc_rows, table, *args):
    nin = len(args) - 25
    idx_refs = args[:nin]
    out = args[nin]
    (ebr, ebc, st_s, st_d, zb, acc, semg, sems) = args[nin + 1:nin + 9]
    rbufs = args[nin + 9:nin + 17]
    sixb = args[nin + 17:nin + 25]

    core = lax.axis_index("c")
    sub = lax.axis_index("s")
    zeros16 = jnp.zeros((16,), jnp.float32)

    def zi(k, _):
        zb[k, pl.ds(0, 16)] = zeros16
        return 0
    lax.fori_loop(0, _ZR, zi, 0)

    for p in range(2):
        slab = 2 * core + p
        for (ridx, cidx, bidir, nrowp, wb_base, d0, s0, d1, s1) in phases:
            rps = nrowp // _NSUB
            pad_slot = nrowp
            r_ref = idx_refs[ridx]
            c_ref = idx_refs[cidx]
            epb = 2 * _B if bidir else _B
            ngrp = epb // _G

            def za(q, _):
                pltpu.sync_copy(zb, acc.at[pl.ds(sub * rps + q * _ZR, _ZR)])
                return 0
            lax.fori_loop(0, rps // _ZR, za, 0)

            @pl.when(sub == 0)
            def _():
                pltpu.sync_copy(zb.at[pl.ds(0, 8)],
                                acc.at[pl.ds(pad_slot, 8)])
            plsc.subcore_barrier()

            eps_ = r_ref.shape[0] // _NSUB
            base = sub * eps_

            def blk(b, _, r_ref=r_ref, c_ref=c_ref, base=base, bidir=bidir,
                    d0=d0, s0=s0, d1=d1, s1=s1, pad_slot=pad_slot,
                    ngrp=ngrp, slab=slab):
                pltpu.sync_copy(r_ref.at[pl.ds(base + b * _B, _B)], ebr)
                pltpu.sync_copy(c_ref.at[pl.ds(base + b * _B, _B)], ebc)

                def vreg(k, _2):
                    r = ebr[pl.ds(k * 16, 16)]
                    c = ebc[pl.ds(k * 16, 16)]
                    ok = r < _SENT
                    if bidir:
                        st_d[pl.ds(32 * k, 16)] = jnp.where(
                            ok, r + d0, pad_slot)
                        st_s[pl.ds(32 * k, 16)] = jnp.where(ok, c + s0, 0)
                        st_d[pl.ds(32 * k + 16, 16)] = jnp.where(
                            ok, c + d1, pad_slot)
                        st_s[pl.ds(32 * k + 16, 16)] = jnp.where(
                            ok, r + s1, 0)
                    else:
                        st_d[pl.ds(16 * k, 16)] = jnp.where(
                            ok, r + d0, pad_slot)
                        st_s[pl.ds(16 * k, 16)] = jnp.where(ok, c + s0, 0)
                    return 0
                lax.fori_loop(0, _B // 16, vreg, 0)

                for w in range(ngrp // 8):
                    gets = []
                    for g in range(8):
                        goff = (w * 8 + g) * _G
                        gets.append(pltpu.async_copy(
                            table.at[slab].at[st_s.at[pl.ds(goff, _G)]],
                            rbufs[g], semg))
                    for g in range(8):
                        goff = (w * 8 + g) * _G
                        for t in range(_G // 16):
                            sixb[g][pl.ds(t * 16, 16)] = st_d[
                                pl.ds(goff + t * 16, 16)]
                    for g in range(8):
                        gets[g].wait()
                    puts = []
                    for g in range(8):
                        puts.append(pltpu.async_copy(
                            rbufs[g], acc.at[sixb[g]], sems, add=True))
                    for g in range(8):
                        puts[g].wait()
                return 0
            lax.fori_loop(0, eps_ // _B, blk, 0)

            plsc.subcore_barrier()
            pltpu.sync_copy(
                acc.at[pl.ds(sub * rps, rps)],
                out.at[slab].at[pl.ds(wb_base + sub * rps, rps)])
            plsc.subcore_barrier()


def _sc_spmm(table3, edge_arrays, phases, acc_rows, n_out):
    mesh = plsc.VectorSubcoreMesh(core_axis_name="c", subcore_axis_name="s")
    k = pl.kernel(
        functools.partial(_spmm_body, phases, acc_rows),
        out_type=jax.ShapeDtypeStruct((_NSLAB, n_out, _W), jnp.float32),
        mesh=mesh,
        compiler_params=pltpu.CompilerParams(use_tc_tiling_on_sc=False),
        scratch_types=[
            pltpu.VMEM((_B,), jnp.int32),
            pltpu.VMEM((_B,), jnp.int32),
            pltpu.VMEM((2 * _B,), jnp.int32),
            pltpu.VMEM((2 * _B,), jnp.int32),
            pltpu.VMEM((_ZR, _W), jnp.float32),
            pltpu.VMEM_SHARED((acc_rows, _W), jnp.float32),
            pltpu.SemaphoreType.DMA,
            pltpu.SemaphoreType.DMA,
        ] + [pltpu.VMEM((_G, _W), jnp.float32)] * 8
          + [pltpu.VMEM((_G,), jnp.int32)] * 8,
    )
    return k(table3, *edge_arrays)


def _deg_body(ui_u, ui_i, ub_u, ub_b, bi_b, out, st_d, ebr, onesb, zb, acc,
              sems, *sixb):
    core = lax.axis_index("c")
    sub = lax.axis_index("s")
    zeros16 = jnp.zeros((16,), jnp.float32)
    ones16 = jnp.ones((16,), jnp.float32)

    def zi(k, _):
        zb[k, pl.ds(0, 16)] = zeros16
        return 0
    lax.fori_loop(0, _ZR, zi, 0)

    def oi(k, _):
        onesb[k, pl.ds(0, 16)] = ones16
        return 0
    lax.fori_loop(0, _G, oi, 0)

    def za(q, _):
        pltpu.sync_copy(zb, acc.at[pl.ds(sub * _DEG_RPS + q * _ZR, _ZR)])
        return 0
    lax.fori_loop(0, _DEG_RPS // _ZR, za, 0)

    @pl.when(sub == 0)
    def _():
        pltpu.sync_copy(zb.at[pl.ds(0, 8)], acc.at[pl.ds(_DEG_ROWS, 8)])
    plsc.subcore_barrier()

    def count(idx_ref, seg_off):
        eps_ = idx_ref.shape[0] // _NSUB
        base = sub * eps_

        def blk(b, _, idx_ref=idx_ref, seg_off=seg_off, base=base):
            pltpu.sync_copy(idx_ref.at[pl.ds(base + b * _B, _B)], ebr)

            def vreg(k, _2):
                v = ebr[pl.ds(k * 16, 16)]
                st_d[pl.ds(16 * k, 16)] = jnp.where(
                    v < _SENT, v + seg_off, _DEG_ROWS)
                return 0
            lax.fori_loop(0, _B // 16, vreg, 0)

            for w in range(_B // _G // 8):
                for g in range(8):
                    goff = (w * 8 + g) * _G
                    for t in range(_G // 16):
                        sixb[g][pl.ds(t * 16, 16)] = st_d[
                            pl.ds(goff + t * 16, 16)]
                puts = []
                for g in range(8):
                    puts.append(pltpu.async_copy(
                        onesb, acc.at[sixb[g]], sems, add=True))
                for pp in puts:
                    pp.wait()
            return 0
        lax.fori_loop(0, eps_ // _B, blk, 0)

    @pl.when(core == 0)
    def _():
        count(ui_u, _DSEG0[0])
        count(ui_i, _DSEG0[1])

    @pl.when(core == 1)
    def _():
        count(ub_u, _DSEG1[0])
        count(ub_b, _DSEG1[1])
        count(bi_b, _DSEG1[2])

    plsc.subcore_barrier()
    pltpu.sync_copy(acc.at[pl.ds(sub * _DEG_RPS, _DEG_RPS)],
                    out.at[core].at[pl.ds(sub * _DEG_RPS, _DEG_RPS)])


def _sc_degrees(ui_u, ui_i, ub_u, ub_b, bi_b):
    mesh = plsc.VectorSubcoreMesh(core_axis_name="c", subcore_axis_name="s")
    k = pl.kernel(
        _deg_body,
        out_type=jax.ShapeDtypeStruct((2, _DEG_ROWS, _W), jnp.float32),
        mesh=mesh,
        scratch_types=[
            pltpu.VMEM((_B,), jnp.int32),
            pltpu.VMEM((_B,), jnp.int32),
            pltpu.VMEM((_G, _W), jnp.float32),
            pltpu.VMEM((_ZR, _W), jnp.float32),
            pltpu.VMEM_SHARED((_DEG_ROWS + 8, _W), jnp.float32),
            pltpu.SemaphoreType.DMA,
        ] + [pltpu.VMEM((_G,), jnp.int32)] * 8,
    )
    return k(ui_u, ui_i, ub_u, ub_b, bi_b)


def _to_slabs(x):
    return x.reshape(x.shape[0], _NSLAB, _W).transpose(1, 0, 2)


def _from_slabs(x3):
    return x3.transpose(1, 0, 2).reshape(x3.shape[1], _D)


_PHASES_MAIN = [
    (0, 1, True, 90112, 0, 0, _OFF_I, _OFF_I, 0),
    (2, 3, True, 73728, _OFF_U2, 0, _OFF_B, _NU, _OFF_U2),
]

_PHASES_BI = [
    (0, 1, False, 24576, 0, 0, _OFF_I, None, None),
]


def kernel(users_feature, bundles_feature, items_feature, ui_u, ui_i, ub_u, ub_b, bi_b, bi_i):
    ui_u, ui_i = _pad_edges(ui_u), _pad_edges(ui_i)
    ub_u, ub_b = _pad_edges(ub_u), _pad_edges(ub_b)
    bi_b, bi_i = _pad_edges(bi_b), _pad_edges(bi_i)

    _USE_SC_DEG = False
    if _USE_SC_DEG:
        degw = _sc_degrees(ui_u, ui_i, ub_u, ub_b, bi_b)
        d0 = degw[0, :, 0]
        d1 = degw[1, :, 0]
    else:
        d0 = jnp.zeros((_DEG_ROWS,), jnp.float32)
        d0 = d0.at[0:_NU].set(jnp.bincount(ui_u[:1000000], length=_NU).astype(jnp.float32))
        d0 = d0.at[_DSEG0[1]:_DSEG0[1] + _NI].set(jnp.bincount(ui_i[:1000000], length=_NI).astype(jnp.float32))
        d1 = jnp.zeros((_DEG_ROWS,), jnp.float32)
        d1 = d1.at[0:_NU].set(jnp.bincount(ub_u[:500000], length=_NU).astype(jnp.float32))
        d1 = d1.at[_DSEG1[1]:_DSEG1[1] + _NB].set(jnp.bincount(ub_b[:500000], length=_NB).astype(jnp.float32))
        d1 = d1.at[_DSEG1[2]:_DSEG1[2] + _NB].set(jnp.bincount(bi_b[:400000], length=_NB).astype(jnp.float32))
    s_all = _ew_1d(_scale_body, jnp.concatenate([
        d0[0:_NU], d0[_DSEG0[1]:_DSEG0[1] + _NI],
        d1[0:_NU], d1[_DSEG1[1]:_DSEG1[1] + _NB],
        jnp.zeros((_NPAD - _NTOT,), jnp.float32),
    ]))
    inv_bs = _ew_1d(_inv_body, jnp.concatenate([
        d1[_DSEG1[2]:_DSEG1[2] + _NB],
        jnp.zeros((24576 - _NB,), jnp.float32),
    ]), rows=192)

    table0 = jnp.concatenate([
        users_feature, items_feature, users_feature, bundles_feature,
        jnp.zeros((_NPAD - _NTOT, _D), jnp.float32),
    ])
    acc = table0
    g = _rowscale(table0, s_all)

    for i in range(2):
        h3 = _sc_spmm(_to_slabs(g), [ui_u, ui_i, ub_u, ub_b], _PHASES_MAIN,
                      acc_rows=90120, n_out=_NPAD)
        acc, g = _layer_update(_from_slabs(h3), s_all, acc, 1.0 / (i + 2))

    hb3 = _sc_spmm(_to_slabs(acc), [bi_b, bi_i], _PHASES_BI,
                   acc_rows=24584, n_out=24576)
    il_bundles = _rowscale(_from_slabs(hb3), inv_bs)[:_NB]

    return (acc[:_NU], acc[_OFF_U2:_OFF_U2 + _NU], il_bundles,
            acc[_OFF_B:_OFF_B + _NB])


# speedup vs baseline: 6.5820x; 1.0843x over previous
import functools
import jax
import jax.numpy as jnp
from jax import lax
from jax.experimental import pallas as pl
from jax.experimental.pallas import tpu as pltpu
from jax.experimental.pallas import tpu_sc as plsc

_NU, _NI, _NB, _D = 50000, 40000, 20000, 64
_OFF_I = _NU
_OFF_U2 = _NU + _NI
_OFF_B = _OFF_U2 + _NU
_NTOT = _OFF_B + _NB
_NPAD = 163840
_EPS = 1e-8
_SENT = 1 << 28
_B = 2048
_G = 128
_ZR = 512
_NSUB = 16
_NSLAB = 4
_W = 16

_DSEG0 = (0, 50048)
_DSEG1 = (0, 50048, 70080)
_DEG_ROWS = 98304
_DEG_RPS = _DEG_ROWS // _NSUB


def _pad_edges(x, blk=_B * _NSUB):
    e = x.shape[0]
    ep = ((e + blk - 1) // blk) * blk
    return jnp.concatenate([x, jnp.full((ep - e,), _SENT, jnp.int32)])


def _scale_body(d_ref, o_ref):
    o_ref[...] = 1.0 / (jnp.sqrt(d_ref[...]) + _EPS)


def _inv_body(d_ref, o_ref):
    o_ref[...] = 1.0 / (d_ref[...] + _EPS)


def _ew_1d(body, x, rows=128):
    n = x.shape[0]
    assert (n // 128) % rows == 0, n
    x2 = x.reshape(n // 128, 128)
    out = pl.pallas_call(
        body,
        out_shape=jax.ShapeDtypeStruct((n // 128, 128), jnp.float32),
        grid=(n // 128 // rows,),
        in_specs=[pl.BlockSpec((rows, 128), lambda i: (i, 0))],
        out_specs=pl.BlockSpec((rows, 128), lambda i: (i, 0)),
    )(x2)
    return out.reshape(n)


def _mul_body(x_ref, s_ref, o_ref):
    o_ref[...] = x_ref[...] * s_ref[...]


def _rowscale(x, s, rows=512):
    n = x.shape[0]
    return pl.pallas_call(
        _mul_body,
        out_shape=jax.ShapeDtypeStruct((n, _D), jnp.float32),
        grid=(n // rows,),
        in_specs=[pl.BlockSpec((rows, _D), lambda i: (i, 0)),
                  pl.BlockSpec((rows, 1), lambda i: (i, 0))],
        out_specs=pl.BlockSpec((rows, _D), lambda i: (i, 0)),
    )(x, s.reshape(n, 1))


def _layer_body(inv_l, h_ref, s_ref, acc_ref, acc_o_ref, g_o_ref):
    s = s_ref[...]
    f = h_ref[...] * s * inv_l
    nrm = jnp.maximum(jnp.sqrt(jnp.sum(f * f, axis=1, keepdims=True)), 1e-12)
    acc_o_ref[...] = acc_ref[...] + f / nrm
    g_o_ref[...] = f * s


def _layer_update(h, s, acc, inv_l, rows=512):
    n = h.shape[0]
    return pl.pallas_call(
        functools.partial(_layer_body, inv_l),
        out_shape=(jax.ShapeDtypeStruct((n, _D), jnp.float32),
                   jax.ShapeDtypeStruct((n, _D), jnp.float32)),
        grid=(n // rows,),
        in_specs=[pl.BlockSpec((rows, _D), lambda i: (i, 0)),
                  pl.BlockSpec((rows, 1), lambda i: (i, 0)),
                  pl.BlockSpec((rows, _D), lambda i: (i, 0))],
        out_specs=(pl.BlockSpec((rows, _D), lambda i: (i, 0)),
                   pl.BlockSpec((rows, _D), lambda i: (i, 0))),
    )(h, s.reshape(n, 1), acc)


def _spmm_body(phases, acc_rows, table, *args):
    nin = len(args) - 25
    idx_refs = args[:nin]
    out = args[nin]
    (ebr, ebc, st_s, st_d, zb, acc, semg, sems) = args[nin + 1:nin + 9]
    rbufs = args[nin + 9:nin + 17]
    sixb = args[nin + 17:nin + 25]

    core = lax.axis_index("c")
    sub = lax.axis_index("s")
    zeros16 = jnp.zeros((16,), jnp.float32)

    def zi(k, _):
        zb[k, pl.ds(0, 16)] = zeros16
        return 0
    lax.fori_loop(0, _ZR, zi, 0)

    for p in range(2):
        slab = 2 * core + p
        for (ridx, cidx, bidir, nrowp, wb_base, d0, s0, d1, s1) in phases:
            rps = nrowp // _NSUB
            pad_slot = nrowp
            r_ref = idx_refs[ridx]
            c_ref = idx_refs[cidx]
            epb = 2 * _B if bidir else _B
            ngrp = epb // _G

            def za(q, _):
                pltpu.sync_copy(zb, acc.at[pl.ds(sub * rps + q * _ZR, _ZR)])
                return 0
            lax.fori_loop(0, rps // _ZR, za, 0)

            @pl.when(sub == 0)
            def _():
                pltpu.sync_copy(zb.at[pl.ds(0, 8)],
                                acc.at[pl.ds(pad_slot, 8)])
            plsc.subcore_barrier()

            eps_ = r_ref.shape[0] // _NSUB
            base = sub * eps_

            def blk(b, _, r_ref=r_ref, c_ref=c_ref, base=base, bidir=bidir,
                    d0=d0, s0=s0, d1=d1, s1=s1, pad_slot=pad_slot,
                    ngrp=ngrp, slab=slab):
                pltpu.sync_copy(r_ref.at[pl.ds(base + b * _B, _B)], ebr)
                pltpu.sync_copy(c_ref.at[pl.ds(base + b * _B, _B)], ebc)

                def vreg(k, _2):
                    r = ebr[pl.ds(k * 16, 16)]
                    c = ebc[pl.ds(k * 16, 16)]
                    ok = r < _SENT
                    if bidir:
                        st_d[pl.ds(32 * k, 16)] = jnp.where(
                            ok, r + d0, pad_slot)
                        st_s[pl.ds(32 * k, 16)] = jnp.where(ok, c + s0, 0)
                        st_d[pl.ds(32 * k + 16, 16)] = jnp.where(
                            ok, c + d1, pad_slot)
                        st_s[pl.ds(32 * k + 16, 16)] = jnp.where(
                            ok, r + s1, 0)
                    else:
                        st_d[pl.ds(16 * k, 16)] = jnp.where(
                            ok, r + d0, pad_slot)
                        st_s[pl.ds(16 * k, 16)] = jnp.where(ok, c + s0, 0)
                    return 0
                lax.fori_loop(0, _B // 16, vreg, 0)

                gets = [None] * ngrp
                puts = [None] * ngrp
                for q in range(ngrp + 4):
                    if q < ngrp:
                        sl = q % 8
                        if q >= 8:
                            puts[q - 8].wait()
                        gets[q] = pltpu.async_copy(
                            table.at[slab].at[st_s.at[pl.ds(q * _G, _G)]],
                            rbufs[sl], semg)
                        for t in range(_G // 16):
                            sixb[sl][pl.ds(t * 16, 16)] = st_d[
                                pl.ds(q * _G + t * 16, 16)]
                    if q >= 4:
                        j = q - 4
                        sl2 = j % 8
                        gets[j].wait()
                        puts[j] = pltpu.async_copy(
                            rbufs[sl2], acc.at[sixb[sl2]], sems, add=True)
                for j in range(ngrp - 8, ngrp):
                    puts[j].wait()
                return 0
            lax.fori_loop(0, eps_ // _B, blk, 0)

            plsc.subcore_barrier()
            pltpu.sync_copy(
                acc.at[pl.ds(sub * rps, rps)],
                out.at[slab].at[pl.ds(wb_base + sub * rps, rps)])
            plsc.subcore_barrier()


def _sc_spmm(table3, edge_arrays, phases, acc_rows, n_out):
    mesh = plsc.VectorSubcoreMesh(core_axis_name="c", subcore_axis_name="s")
    k = pl.kernel(
        functools.partial(_spmm_body, phases, acc_rows),
        out_type=jax.ShapeDtypeStruct((_NSLAB, n_out, _W), jnp.float32),
        mesh=mesh,
        compiler_params=pltpu.CompilerParams(use_tc_tiling_on_sc=False),
        scratch_types=[
            pltpu.VMEM((_B,), jnp.int32),
            pltpu.VMEM((_B,), jnp.int32),
            pltpu.VMEM((2 * _B,), jnp.int32),
            pltpu.VMEM((2 * _B,), jnp.int32),
            pltpu.VMEM((_ZR, _W), jnp.float32),
            pltpu.VMEM_SHARED((acc_rows, _W), jnp.float32),
            pltpu.SemaphoreType.DMA,
            pltpu.SemaphoreType.DMA,
        ] + [pltpu.VMEM((_G, _W), jnp.float32)] * 8
          + [pltpu.VMEM((_G,), jnp.int32)] * 8,
    )
    return k(table3, *edge_arrays)


def _deg_body(ui_u, ui_i, ub_u, ub_b, bi_b, out, st_d, ebr, onesb, zb, acc,
              sems, *sixb):
    core = lax.axis_index("c")
    sub = lax.axis_index("s")
    zeros16 = jnp.zeros((16,), jnp.float32)
    ones16 = jnp.ones((16,), jnp.float32)

    def zi(k, _):
        zb[k, pl.ds(0, 16)] = zeros16
        return 0
    lax.fori_loop(0, _ZR, zi, 0)

    def oi(k, _):
        onesb[k, pl.ds(0, 16)] = ones16
        return 0
    lax.fori_loop(0, _G, oi, 0)

    def za(q, _):
        pltpu.sync_copy(zb, acc.at[pl.ds(sub * _DEG_RPS + q * _ZR, _ZR)])
        return 0
    lax.fori_loop(0, _DEG_RPS // _ZR, za, 0)

    @pl.when(sub == 0)
    def _():
        pltpu.sync_copy(zb.at[pl.ds(0, 8)], acc.at[pl.ds(_DEG_ROWS, 8)])
    plsc.subcore_barrier()

    def count(idx_ref, seg_off):
        eps_ = idx_ref.shape[0] // _NSUB
        base = sub * eps_

        def blk(b, _, idx_ref=idx_ref, seg_off=seg_off, base=base):
            pltpu.sync_copy(idx_ref.at[pl.ds(base + b * _B, _B)], ebr)

            def vreg(k, _2):
                v = ebr[pl.ds(k * 16, 16)]
                st_d[pl.ds(16 * k, 16)] = jnp.where(
                    v < _SENT, v + seg_off, _DEG_ROWS)
                return 0
            lax.fori_loop(0, _B // 16, vreg, 0)

            for w in range(_B // _G // 8):
                for g in range(8):
                    goff = (w * 8 + g) * _G
                    for t in range(_G // 16):
                        sixb[g][pl.ds(t * 16, 16)] = st_d[
                            pl.ds(goff + t * 16, 16)]
                puts = []
                for g in range(8):
                    puts.append(pltpu.async_copy(
                        onesb, acc.at[sixb[g]], sems, add=True))
                for pp in puts:
                    pp.wait()
            return 0
        lax.fori_loop(0, eps_ // _B, blk, 0)

    @pl.when(core == 0)
    def _():
        count(ui_u, _DSEG0[0])
        count(ui_i, _DSEG0[1])

    @pl.when(core == 1)
    def _():
        count(ub_u, _DSEG1[0])
        count(ub_b, _DSEG1[1])
        count(bi_b, _DSEG1[2])

    plsc.subcore_barrier()
    pltpu.sync_copy(acc.at[pl.ds(sub * _DEG_RPS, _DEG_RPS)],
                    out.at[core].at[pl.ds(sub * _DEG_RPS, _DEG_RPS)])


def _sc_degrees(ui_u, ui_i, ub_u, ub_b, bi_b):
    mesh = plsc.VectorSubcoreMesh(core_axis_name="c", subcore_axis_name="s")
    k = pl.kernel(
        _deg_body,
        out_type=jax.ShapeDtypeStruct((2, _DEG_ROWS, _W), jnp.float32),
        mesh=mesh,
        scratch_types=[
            pltpu.VMEM((_B,), jnp.int32),
            pltpu.VMEM((_B,), jnp.int32),
            pltpu.VMEM((_G, _W), jnp.float32),
            pltpu.VMEM((_ZR, _W), jnp.float32),
            pltpu.VMEM_SHARED((_DEG_ROWS + 8, _W), jnp.float32),
            pltpu.SemaphoreType.DMA,
        ] + [pltpu.VMEM((_G,), jnp.int32)] * 8,
    )
    return k(ui_u, ui_i, ub_u, ub_b, bi_b)


def _to_slabs(x):
    return x.reshape(x.shape[0], _NSLAB, _W).transpose(1, 0, 2)


def _from_slabs(x3):
    return x3.transpose(1, 0, 2).reshape(x3.shape[1], _D)


_PHASES_MAIN = [
    (0, 1, True, 90112, 0, 0, _OFF_I, _OFF_I, 0),
    (2, 3, True, 73728, _OFF_U2, 0, _OFF_B, _NU, _OFF_U2),
]

_PHASES_BI = [
    (0, 1, False, 24576, 0, 0, _OFF_I, None, None),
]


def kernel(users_feature, bundles_feature, items_feature, ui_u, ui_i, ub_u, ub_b, bi_b, bi_i):
    ui_u, ui_i = _pad_edges(ui_u), _pad_edges(ui_i)
    ub_u, ub_b = _pad_edges(ub_u), _pad_edges(ub_b)
    bi_b, bi_i = _pad_edges(bi_b), _pad_edges(bi_i)

    _USE_SC_DEG = False
    if _USE_SC_DEG:
        degw = _sc_degrees(ui_u, ui_i, ub_u, ub_b, bi_b)
        d0 = degw[0, :, 0]
        d1 = degw[1, :, 0]
    else:
        d0 = jnp.zeros((_DEG_ROWS,), jnp.float32)
        d0 = d0.at[0:_NU].set(jnp.bincount(ui_u[:1000000], length=_NU).astype(jnp.float32))
        d0 = d0.at[_DSEG0[1]:_DSEG0[1] + _NI].set(jnp.bincount(ui_i[:1000000], length=_NI).astype(jnp.float32))
        d1 = jnp.zeros((_DEG_ROWS,), jnp.float32)
        d1 = d1.at[0:_NU].set(jnp.bincount(ub_u[:500000], length=_NU).astype(jnp.float32))
        d1 = d1.at[_DSEG1[1]:_DSEG1[1] + _NB].set(jnp.bincount(ub_b[:500000], length=_NB).astype(jnp.float32))
        d1 = d1.at[_DSEG1[2]:_DSEG1[2] + _NB].set(jnp.bincount(bi_b[:400000], length=_NB).astype(jnp.float32))
    s_all = _ew_1d(_scale_body, jnp.concatenate([
        d0[0:_NU], d0[_DSEG0[1]:_DSEG0[1] + _NI],
        d1[0:_NU], d1[_DSEG1[1]:_DSEG1[1] + _NB],
        jnp.zeros((_NPAD - _NTOT,), jnp.float32),
    ]))
    inv_bs = _ew_1d(_inv_body, jnp.concatenate([
        d1[_DSEG1[2]:_DSEG1[2] + _NB],
        jnp.zeros((24576 - _NB,), jnp.float32),
    ]), rows=192)

    table0 = jnp.concatenate([
        users_feature, items_feature, users_feature, bundles_feature,
        jnp.zeros((_NPAD - _NTOT, _D), jnp.float32),
    ])
    acc = table0
    g = _rowscale(table0, s_all)

    for i in range(2):
        h3 = _sc_spmm(_to_slabs(g), [ui_u, ui_i, ub_u, ub_b], _PHASES_MAIN,
                      acc_rows=90120, n_out=_NPAD)
        acc, g = _layer_update(_from_slabs(h3), s_all, acc, 1.0 / (i + 2))

    hb3 = _sc_spmm(_to_slabs(acc), [bi_b, bi_i], _PHASES_BI,
                   acc_rows=24584, n_out=24576)
    il_bundles = _rowscale(_from_slabs(hb3), inv_bs)[:_NB]

    return (acc[:_NU], acc[_OFF_U2:_OFF_U2 + _NU], il_bundles,
            acc[_OFF_B:_OFF_B + _NB])
